# baseline (speedup 1.0000x reference)
import jax
import jax.numpy as jnp
from jax import lax
from jax.experimental import pallas as pl
from jax.experimental.pallas import tpu as pltpu
from jax.experimental.pallas import tpu_sc as plsc

_VOCAB = 28
_N_ALPHA = 20
_ALPHA_OFFSET = 3
_BLOSUM_DIM = 24

_NC = 2
_NS = 16
_NW = _NC * _NS
_LANES = 16


def _sc_gather_body(src_hbm, table_hbm, out_hbm, idx_v, table_v, col_v, sem):
    del sem
    ntok = idx_v.shape[0]
    wid = lax.axis_index("s") * _NC + lax.axis_index("c")
    base = wid * ntok
    pltpu.sync_copy(table_hbm, table_v.at[pl.ds(0, _VOCAB * _BLOSUM_DIM)])
    pltpu.sync_copy(src_hbm.at[pl.ds(base, ntok)], idx_v)

    def gather(i, carry):
        v = idx_v[pl.ds(i * _LANES, _LANES)]
        valid = (v >= _ALPHA_OFFSET) & (v < _ALPHA_OFFSET + _N_ALPHA)
        off = jnp.where(valid, v, _VOCAB - 1) * _BLOSUM_DIM
        for j in range(_BLOSUM_DIM):
            col_v[pl.ds(j * ntok + i * _LANES, _LANES)] = plsc.load_gather(
                table_v, [off + j]
            )
        return carry

    lax.fori_loop(0, ntok // _LANES, gather, 0)

    for j in range(_BLOSUM_DIM):
        pltpu.sync_copy(
            col_v.at[pl.ds(j * ntok, ntok)],
            out_hbm.at[j, pl.ds(base, ntok)],
        )


def _sc_gather(srcf, tablef):
    n = srcf.shape[0]
    ntok = n // _NW
    mesh = plsc.VectorSubcoreMesh(core_axis_name="c", subcore_axis_name="s")
    f = pl.kernel(
        _sc_gather_body,
        out_type=jax.ShapeDtypeStruct((_BLOSUM_DIM, n), jnp.float32),
        mesh=mesh,
        compiler_params=pltpu.CompilerParams(needs_layout_passes=False),
        scratch_types=[
            pltpu.VMEM((ntok,), jnp.int32),
            pltpu.VMEM((1024,), jnp.float32),
            pltpu.VMEM((_BLOSUM_DIM * ntok,), jnp.float32),
            pltpu.SemaphoreType.DMA,
        ],
    )
    return f(srcf, tablef)


def _tc_concat_body(x_ref, cod_ref, out_ref):
    bm, ln, d = x_ref.shape
    codt = cod_ref[...]
    n = codt.shape[0]
    eye = (
        lax.broadcasted_iota(jnp.int32, (n, n), 0)
        == lax.broadcasted_iota(jnp.int32, (n, n), 1)
    ).astype(jnp.float32)
    for i in range(bm):
        cod = lax.dot_general(
            codt[:, i * ln:(i + 1) * ln], eye, (((0,), (0,)), ((), ())),
            precision=lax.Precision.HIGHEST,
            preferred_element_type=jnp.float32,
        )
        out_ref[i] = jnp.concatenate([x_ref[i], cod], axis=1)


def kernel(src, x, blosum):
    B, L, D = x.shape
    tablef = blosum.reshape(_VOCAB * _BLOSUM_DIM)
    srcf = src.astype(jnp.int32).reshape(B * L)
    coding = _sc_gather(srcf, tablef)
    BM = 4
    out = pl.pallas_call(
        _tc_concat_body,
        grid=(B // BM,),
        in_specs=[
            pl.BlockSpec((BM, L, D), lambda b: (b, 0, 0)),
            pl.BlockSpec((_BLOSUM_DIM, BM * L), lambda b: (0, b)),
        ],
        out_specs=pl.BlockSpec((BM, L, D + _BLOSUM_DIM), lambda b: (b, 0, 0)),
        out_shape=jax.ShapeDtypeStruct((B, L, D + _BLOSUM_DIM), jnp.float32),
    )(x, coding)
    return out
